# baseline (speedup 1.0000x reference)
import functools

import jax
import jax.numpy as jnp
from jax import lax
from jax.experimental import pallas as pl
from jax.experimental.pallas import tpu as pltpu
from jax.experimental.pallas import tpu_sc as plsc

N_NODES = 10000
N_PAD = 10240
N_EDGES = 320000

NUM_CORES = 2
NUM_SUBCORES = 16
NUM_WORKERS = NUM_CORES * NUM_SUBCORES

CHUNK = 80
CHUNKS_PER_WORKER = N_EDGES // CHUNK // NUM_WORKERS
ROWS_PER_TILE = N_PAD // NUM_SUBCORES
ZCOPIES = ROWS_PER_TILE // CHUNK


NBUF = 2
NHALF = 2
HC = CHUNK // NHALF


def _sc_agg_kernel(feat_hbm, src_hbm, dst4_hbm, zrows_hbm, ones_hbm,
                   out_hbm, deg_hbm,
                   acc_sh, src_all, dst3_v, rows, sg, ss,
                   *, d, with_deg):
    cid = lax.axis_index("c")
    sid = lax.axis_index("s")
    wid = cid * NUM_SUBCORES + sid

    r0 = sid * ROWS_PER_TILE
    pltpu.sync_copy(zrows_hbm, rows[0])
    for k in range(ZCOPIES):
        pltpu.sync_copy(rows[0],
                        acc_sh.at[pl.ds(r0 + k * CHUNK, CHUNK), :])
    del r0

    e0 = wid * CHUNKS_PER_WORKER * CHUNK
    pltpu.sync_copy(src_hbm.at[pl.ds(e0, CHUNKS_PER_WORKER * CHUNK)],
                    src_all)
    pltpu.sync_copy(dst4_hbm.at[wid], dst3_v)
    plsc.subcore_barrier()

    nsteps = CHUNKS_PER_WORKER // NBUF

    def body(t, _):
        j0 = NBUF * t
        gs = [pltpu.async_copy(
                  feat_hbm.at[src_all.at[pl.ds((j0 + b) * CHUNK, CHUNK)]],
                  rows[b], sg[b])
              for b in range(NBUF)]
        scs = []
        for b in range(NBUF):
            gs[b].wait()
            scs.append(pltpu.async_copy(
                rows[b], acc_sh.at[dst3_v.at[j0 + b]], ss[b], add=True))
        for s in scs:
            s.wait()
        return 0

    lax.fori_loop(0, nsteps, body, 0)
    for j in range(nsteps * NBUF, CHUNKS_PER_WORKER):
        pltpu.async_copy(
            feat_hbm.at[src_all.at[pl.ds(j * CHUNK, CHUNK)]],
            rows[0], sg[0]).wait()
        pltpu.async_copy(rows[0], acc_sh.at[dst3_v.at[j]], ss[0],
                         add=True).wait()
    plsc.subcore_barrier()
    r0 = sid * ROWS_PER_TILE

    o0 = cid * N_PAD + r0
    for k in range(ZCOPIES):
        pltpu.sync_copy(acc_sh.at[pl.ds(r0 + k * CHUNK, CHUNK), :],
                        rows[k % NBUF])
        pltpu.sync_copy(rows[k % NBUF],
                        out_hbm.at[pl.ds(o0 + k * CHUNK, CHUNK), :])

    if with_deg:
        pltpu.sync_copy(zrows_hbm, rows[0])
        for k in range(ZCOPIES):
            pltpu.sync_copy(rows[0],
                            acc_sh.at[pl.ds(r0 + k * CHUNK, CHUNK), :])
        pltpu.sync_copy(ones_hbm, rows[1])
        plsc.subcore_barrier()

        def dbody(t, _):
            j0 = NBUF * t
            scs = [pltpu.async_copy(
                       rows[1], acc_sh.at[dst3_v.at[j0 + b]], ss[b],
                       add=True)
                   for b in range(NBUF)]
            for s in scs:
                s.wait()
            return 0

        lax.fori_loop(0, nsteps, dbody, 0)
        for j in range(nsteps * NBUF, CHUNKS_PER_WORKER):
            pltpu.async_copy(rows[1], acc_sh.at[dst3_v.at[j]], ss[0],
                             add=True).wait()
        plsc.subcore_barrier()

        for k in range(ZCOPIES):
            pltpu.sync_copy(acc_sh.at[pl.ds(r0 + k * CHUNK, CHUNK), :],
                            rows[k % NBUF])
            pltpu.sync_copy(rows[k % NBUF],
                            deg_hbm.at[pl.ds(o0 + k * CHUNK, CHUNK), :])


def _make_sc_agg(d, with_deg):
    mesh = plsc.VectorSubcoreMesh(core_axis_name="c", subcore_axis_name="s")
    out_type = [
        jax.ShapeDtypeStruct((NUM_CORES * N_PAD, d), jnp.float32),
        jax.ShapeDtypeStruct((NUM_CORES * N_PAD, d), jnp.float32),
    ]
    scratch = [
        pltpu.VMEM_SHARED((N_PAD, d), jnp.float32),
        pltpu.VMEM((CHUNKS_PER_WORKER * CHUNK,), jnp.int32),
        pltpu.VMEM((CHUNKS_PER_WORKER, CHUNK), jnp.int32),
        [pltpu.VMEM((CHUNK, d), jnp.float32) for _ in range(NBUF)],
        [pltpu.SemaphoreType.DMA for _ in range(NBUF)],
        [pltpu.SemaphoreType.DMA for _ in range(NBUF)],
    ]
    return pl.kernel(
        functools.partial(_sc_agg_kernel, d=d, with_deg=with_deg),
        out_type=out_type,
        mesh=mesh,
        scratch_types=scratch,
    )


def _tc1_kernel(x_ref, s1a_ref, s1b_ref, da_ref, db_ref, w1_ref, b1_ref,
                h_ref):
    rdeg = 1.0 / jnp.maximum(da_ref[...] + db_ref[...], 1.0)
    a1 = (s1a_ref[...] + s1b_ref[...]) * rdeg
    x = x_ref[...]
    h = (jnp.dot(x, w1_ref[:128, :], preferred_element_type=jnp.float32)
         + jnp.dot(a1, w1_ref[128:, :], preferred_element_type=jnp.float32)
         + b1_ref[...])
    h_ref[...] = jnp.maximum(h, 0.0)


def _tc2_kernel(h_ref, s2a_ref, s2b_ref, da_ref, db_ref, w2_ref, b2_ref,
                out_ref):
    rdeg = 1.0 / jnp.maximum(da_ref[...] + db_ref[...], 1.0)
    a2 = (s2a_ref[...] + s2b_ref[...]) * rdeg
    out_ref[...] = (
        jnp.dot(h_ref[...], w2_ref[:128, :], preferred_element_type=jnp.float32)
        + jnp.dot(a2, w2_ref[128:, :], preferred_element_type=jnp.float32)
        + b2_ref[...])


_TC_BLOCK = 1024


def _row_spec(d):
    return pl.BlockSpec((_TC_BLOCK, d), lambda i: (i, 0))


def _full_spec(shape):
    return pl.BlockSpec(shape, lambda i: tuple(0 for _ in shape))


def kernel(X, edge_index, W1, b1, W2, b2):
    src = edge_index[0]
    dst = edge_index[1]
    z128 = jnp.zeros((CHUNK, 128), jnp.float32)

    ones128 = jnp.ones((CHUNK, 128), jnp.float32)
    dst4 = dst.reshape(NUM_WORKERS, CHUNKS_PER_WORKER, CHUNK)

    s1, degp = _make_sc_agg(128, True)(X, src, dst4, z128, ones128)
    s1 = s1.reshape(NUM_CORES, N_PAD, 128)
    degp = degp.reshape(NUM_CORES, N_PAD, 128)
    da = degp[0, :N_NODES, 0:1]
    db = degp[1, :N_NODES, 0:1]

    grid = pl.cdiv(N_NODES, _TC_BLOCK)
    h = pl.pallas_call(
        _tc1_kernel,
        grid=(grid,),
        in_specs=[
            _row_spec(128), _row_spec(128), _row_spec(128), _row_spec(1),
            _row_spec(1),
            _full_spec((256, 128)), _full_spec((1, 128)),
        ],
        out_specs=_row_spec(128),
        out_shape=jax.ShapeDtypeStruct((N_NODES, 128), jnp.float32),
    )(X, s1[0], s1[1], da, db, W1, b1.reshape(1, 128))

    s2, _ = _make_sc_agg(128, False)(h, src, dst4, z128, ones128)
    s2 = s2.reshape(NUM_CORES, N_PAD, 128)

    out = pl.pallas_call(
        _tc2_kernel,
        grid=(grid,),
        in_specs=[
            _row_spec(128), _row_spec(128), _row_spec(128), _row_spec(1),
            _row_spec(1),
            _full_spec((256, 64)), _full_spec((1, 64)),
        ],
        out_specs=_row_spec(64),
        out_shape=jax.ShapeDtypeStruct((N_NODES, 64), jnp.float32),
    )(h, s2[0], s2[1], da, db, W2, b2.reshape(1, 64))

    return out


# speedup vs baseline: 1.0002x; 1.0002x over previous
import functools

import jax
import jax.numpy as jnp
from jax import lax
from jax.experimental import pallas as pl
from jax.experimental.pallas import tpu as pltpu
from jax.experimental.pallas import tpu_sc as plsc

N_NODES = 10000
N_PAD = 10240
N_EDGES = 320000

NUM_CORES = 2
NUM_SUBCORES = 16
NUM_WORKERS = NUM_CORES * NUM_SUBCORES

CHUNK = 80
CHUNKS_PER_WORKER = N_EDGES // CHUNK // NUM_WORKERS
ROWS_PER_TILE = N_PAD // NUM_SUBCORES
ZCOPIES = ROWS_PER_TILE // CHUNK


NBUF = 2


def _sc_agg_kernel(feat_hbm, src_hbm, dst4_hbm, zrows_hbm, ones_hbm,
                   out_hbm, deg_hbm,
                   acc_sh, src_all, dst3_v, rows, sg, ss,
                   *, d, with_deg):
    cid = lax.axis_index("c")
    sid = lax.axis_index("s")
    wid = cid * NUM_SUBCORES + sid

    r0 = sid * ROWS_PER_TILE
    pltpu.sync_copy(zrows_hbm, rows[0])
    for k in range(ZCOPIES):
        pltpu.sync_copy(rows[0],
                        acc_sh.at[pl.ds(r0 + k * CHUNK, CHUNK), :])

    e0 = wid * CHUNKS_PER_WORKER * CHUNK
    pltpu.sync_copy(src_hbm.at[pl.ds(e0, CHUNKS_PER_WORKER * CHUNK)],
                    src_all)
    pltpu.sync_copy(dst4_hbm.at[wid], dst3_v)
    plsc.subcore_barrier()

    nsteps = CHUNKS_PER_WORKER // NBUF

    def body(t, _):
        j0 = NBUF * t
        gs = [pltpu.async_copy(
                  feat_hbm.at[src_all.at[pl.ds((j0 + b) * CHUNK, CHUNK)]],
                  rows[b], sg[b])
              for b in range(NBUF)]
        scs = []
        for b in range(NBUF):
            gs[b].wait()
            scs.append(pltpu.async_copy(
                rows[b], acc_sh.at[dst3_v.at[j0 + b]], ss[b], add=True))
        for s in scs:
            s.wait()
        return 0

    lax.fori_loop(0, nsteps, body, 0)
    for j in range(nsteps * NBUF, CHUNKS_PER_WORKER):
        pltpu.async_copy(
            feat_hbm.at[src_all.at[pl.ds(j * CHUNK, CHUNK)]],
            rows[0], sg[0]).wait()
        pltpu.async_copy(rows[0], acc_sh.at[dst3_v.at[j]], ss[0],
                         add=True).wait()
    plsc.subcore_barrier()
    r0 = sid * ROWS_PER_TILE

    o0 = cid * N_PAD + r0
    for k in range(ZCOPIES):
        pltpu.sync_copy(acc_sh.at[pl.ds(r0 + k * CHUNK, CHUNK), :],
                        rows[k % NBUF])
        pltpu.sync_copy(rows[k % NBUF],
                        out_hbm.at[pl.ds(o0 + k * CHUNK, CHUNK), :])

    if with_deg:
        pltpu.sync_copy(zrows_hbm, rows[0])
        for k in range(ZCOPIES):
            pltpu.sync_copy(rows[0],
                            acc_sh.at[pl.ds(r0 + k * CHUNK, CHUNK), :])
        pltpu.sync_copy(ones_hbm, rows[1])
        plsc.subcore_barrier()

        def dbody(t, _):
            j0 = NBUF * t
            scs = [pltpu.async_copy(
                       rows[1], acc_sh.at[dst3_v.at[j0 + b]], ss[b],
                       add=True)
                   for b in range(NBUF)]
            for s in scs:
                s.wait()
            return 0

        lax.fori_loop(0, nsteps, dbody, 0)
        for j in range(nsteps * NBUF, CHUNKS_PER_WORKER):
            pltpu.async_copy(rows[1], acc_sh.at[dst3_v.at[j]], ss[0],
                             add=True).wait()
        plsc.subcore_barrier()

        for k in range(ZCOPIES):
            pltpu.sync_copy(acc_sh.at[pl.ds(r0 + k * CHUNK, CHUNK), :],
                            rows[k % NBUF])
            pltpu.sync_copy(rows[k % NBUF],
                            deg_hbm.at[pl.ds(o0 + k * CHUNK, CHUNK), :])


def _make_sc_agg(d, with_deg):
    mesh = plsc.VectorSubcoreMesh(core_axis_name="c", subcore_axis_name="s")
    out_type = [
        jax.ShapeDtypeStruct((NUM_CORES * N_PAD, d), jnp.float32),
        jax.ShapeDtypeStruct((NUM_CORES * N_PAD, d), jnp.float32),
    ]
    scratch = [
        pltpu.VMEM_SHARED((N_PAD, d), jnp.float32),
        pltpu.VMEM((CHUNKS_PER_WORKER * CHUNK,), jnp.int32),
        pltpu.VMEM((CHUNKS_PER_WORKER, CHUNK), jnp.int32),
        [pltpu.VMEM((CHUNK, d), jnp.float32) for _ in range(NBUF)],
        [pltpu.SemaphoreType.DMA for _ in range(NBUF)],
        [pltpu.SemaphoreType.DMA for _ in range(NBUF)],
    ]
    return pl.kernel(
        functools.partial(_sc_agg_kernel, d=d, with_deg=with_deg),
        out_type=out_type,
        mesh=mesh,
        scratch_types=scratch,
    )


def _tc1_kernel(x_ref, s1a_ref, s1b_ref, da_ref, db_ref, w1_ref, b1_ref,
                h_ref):
    rdeg = 1.0 / jnp.maximum(da_ref[...] + db_ref[...], 1.0)
    a1 = (s1a_ref[...] + s1b_ref[...]) * rdeg
    x = x_ref[...]
    h = (jnp.dot(x, w1_ref[:128, :], preferred_element_type=jnp.float32)
         + jnp.dot(a1, w1_ref[128:, :], preferred_element_type=jnp.float32)
         + b1_ref[...])
    h_ref[...] = jnp.maximum(h, 0.0)


def _tc2_kernel(h_ref, s2a_ref, s2b_ref, da_ref, db_ref, w2_ref, b2_ref,
                out_ref):
    rdeg = 1.0 / jnp.maximum(da_ref[...] + db_ref[...], 1.0)
    a2 = (s2a_ref[...] + s2b_ref[...]) * rdeg
    out_ref[...] = (
        jnp.dot(h_ref[...], w2_ref[:128, :], preferred_element_type=jnp.float32)
        + jnp.dot(a2, w2_ref[128:, :], preferred_element_type=jnp.float32)
        + b2_ref[...])


_TC_BLOCK = 1024


def _row_spec(d):
    return pl.BlockSpec((_TC_BLOCK, d), lambda i: (i, 0))


def _full_spec(shape):
    return pl.BlockSpec(shape, lambda i: tuple(0 for _ in shape))


def kernel(X, edge_index, W1, b1, W2, b2):
    src = edge_index[0]
    dst = edge_index[1]
    z128 = jnp.zeros((CHUNK, 128), jnp.float32)

    ones128 = jnp.ones((CHUNK, 128), jnp.float32)
    dst4 = dst.reshape(NUM_WORKERS, CHUNKS_PER_WORKER, CHUNK)

    s1, degp = _make_sc_agg(128, True)(X, src, dst4, z128, ones128)
    s1 = s1.reshape(NUM_CORES, N_PAD, 128)
    degp = degp.reshape(NUM_CORES, N_PAD, 128)
    da = degp[0, :N_NODES, 0:1]
    db = degp[1, :N_NODES, 0:1]

    grid = pl.cdiv(N_NODES, _TC_BLOCK)
    h = pl.pallas_call(
        _tc1_kernel,
        grid=(grid,),
        in_specs=[
            _row_spec(128), _row_spec(128), _row_spec(128), _row_spec(1),
            _row_spec(1),
            _full_spec((256, 128)), _full_spec((1, 128)),
        ],
        out_specs=_row_spec(128),
        out_shape=jax.ShapeDtypeStruct((N_NODES, 128), jnp.float32),
    )(X, s1[0], s1[1], da, db, W1, b1.reshape(1, 128))

    s2, _ = _make_sc_agg(128, False)(h, src, dst4, z128, ones128)
    s2 = s2.reshape(NUM_CORES, N_PAD, 128)

    out = pl.pallas_call(
        _tc2_kernel,
        grid=(grid,),
        in_specs=[
            _row_spec(128), _row_spec(128), _row_spec(128), _row_spec(1),
            _row_spec(1),
            _full_spec((256, 64)), _full_spec((1, 64)),
        ],
        out_specs=_row_spec(64),
        out_shape=jax.ShapeDtypeStruct((N_NODES, 64), jnp.float32),
    )(h, s2[0], s2[1], da, db, W2, b2.reshape(1, 64))

    return out
